# baseline (speedup 1.0000x reference)
import functools

import jax
import jax.numpy as jnp
from jax import lax
from jax.experimental import pallas as pl
from jax.experimental.pallas import tpu as pltpu
from jax.experimental.pallas import tpu_sc as plsc

VOCAB = 100000
EMB = 64
N_CTX = 16384

NW = 32
PER_W = N_CTX // NW
CHUNK = 128
NCHUNK = PER_W // CHUNK

VOCAB_PAD = 100352
STRIPE = VOCAB_PAD // 16

BV = 50176
NBLK = VOCAB_PAD // BV


def _hist_body(idx_hbm, out0_hbm, out1_hbm, idx_v, zbuf, ones_v, shared, sem):
    core = lax.axis_index("c")
    sub = lax.axis_index("s")
    wid = sub * 2 + core

    idx_copies = [
        pltpu.async_copy(
            idx_hbm.at[pl.ds(wid * PER_W + k * CHUNK, CHUNK)], idx_v.at[k], sem
        )
        for k in range(NCHUNK)
    ]

    z16 = jnp.zeros((16,), jnp.float32)

    def zero_body(j, _):
        for u in range(8):
            zbuf[pl.ds(j * 128 + u * 16, 16)] = z16
        return 0

    lax.fori_loop(0, STRIPE // 128, zero_body, 0)
    for j in range(CHUNK // 16):
        ones_v[pl.ds(j * 16, 16)] = z16 + 1.0
    pltpu.sync_copy(zbuf, shared.at[pl.ds(sub * STRIPE, STRIPE)])
    for c in idx_copies:
        c.wait()
    plsc.subcore_barrier()
    for k in range(NCHUNK):
        pltpu.sync_copy(ones_v, shared.at[idx_v.at[k]], add=True)
    plsc.subcore_barrier()

    @pl.when(core == 0)
    def _out0():
        pltpu.sync_copy(
            shared.at[pl.ds(sub * STRIPE, STRIPE)],
            out0_hbm.at[pl.ds(sub * STRIPE, STRIPE)],
        )

    @pl.when(core == 1)
    def _out1():
        pltpu.sync_copy(
            shared.at[pl.ds(sub * STRIPE, STRIPE)],
            out1_hbm.at[pl.ds(sub * STRIPE, STRIPE)],
        )


@functools.cache
def _hist():
    return functools.partial(
        pl.kernel,
        out_type=[
            jax.ShapeDtypeStruct((VOCAB_PAD,), jnp.float32),
            jax.ShapeDtypeStruct((VOCAB_PAD,), jnp.float32),
        ],
        mesh=plsc.VectorSubcoreMesh(core_axis_name="c", subcore_axis_name="s"),
        scratch_types=[
            pltpu.VMEM((NCHUNK, CHUNK), jnp.int32),
            pltpu.VMEM((STRIPE,), jnp.float32),
            pltpu.VMEM((CHUNK,), jnp.float32),
            pltpu.VMEM_SHARED((VOCAB_PAD,), jnp.float32),
            pltpu.SemaphoreType.DMA,
        ],
    )(_hist_body)


def _tc_body(c0_ref, c1_ref, t_ref, w_ref, b_ref, out_ref, s_acc, mv, sv, logits_s, sm):
    p = pl.program_id(0)
    i = pl.program_id(1)
    lane = lax.broadcasted_iota(jnp.int32, (1, BV), 1)
    nvalid = VOCAB - i * BV

    @pl.when((p == 0) & (i == 0))
    def _init_s():
        s_acc[...] = jnp.zeros((1, EMB), jnp.float32)

    @pl.when((p == 0) & (i < NBLK - 1))
    def _phase0():
        c = c0_ref[...] + c1_ref[...]
        s_acc[...] += lax.dot_general(
            c, t_ref[...], (((1,), (1,)), ((), ())),
            preferred_element_type=jnp.float32,
        )

    @pl.when((p == 0) & (i == NBLK - 1))
    def _phase0_edge():
        c = c0_ref[...] + c1_ref[...]
        t = jnp.where(lane < nvalid, t_ref[...], 0.0)
        s_acc[...] += lax.dot_general(
            c, t, (((1,), (1,)), ((), ())),
            preferred_element_type=jnp.float32,
        )

    @pl.when((p == 1) & (i == 0))
    def _init_ms():
        mv[...] = jnp.full((1, BV), -1e30, jnp.float32)
        sv[...] = jnp.zeros((1, BV), jnp.float32)

    @pl.when(p == 1)
    def _phase1():
        s2 = s_acc[...] * (1.0 / N_CTX)
        l = lax.dot_general(
            s2, w_ref[...], (((1,), (0,)), ((), ())),
            preferred_element_type=jnp.float32,
        ) + b_ref[...]
        l = jnp.where(lane < nvalid, l, -1e30)
        logits_s[i] = l
        m_old = mv[...]
        m_new = jnp.maximum(m_old, l)
        sv[...] = sv[...] * jnp.exp(m_old - m_new) + jnp.exp(l - m_new)
        mv[...] = m_new

    @pl.when((p == 2) & (i == 0))
    def _logz():
        m = jnp.max(mv[...])
        sm[0] = m + jnp.log(jnp.sum(sv[...] * jnp.exp(mv[...] - m)))

    @pl.when(p == 2)
    def _phase2():
        out_ref[...] = logits_s[i] - sm[0]


def _tc_call(c0, c1, tT, wT, b2):
    last = NBLK - 1
    return pl.pallas_call(
        _tc_body,
        grid=(3, NBLK),
        in_specs=[
            pl.BlockSpec((1, BV), lambda p, i: (0, jnp.where(p == 0, i, last))),
            pl.BlockSpec((1, BV), lambda p, i: (0, jnp.where(p == 0, i, last))),
            pl.BlockSpec((EMB, BV), lambda p, i: (0, jnp.where(p == 0, i, last))),
            pl.BlockSpec((EMB, BV), lambda p, i: (0, jnp.where(p == 1, i, jnp.where(p == 0, 0, last)))),
            pl.BlockSpec((1, BV), lambda p, i: (0, jnp.where(p == 1, i, 0))),
        ],
        out_specs=pl.BlockSpec((1, BV), lambda p, i: (0, jnp.where(p == 2, i, 0))),
        out_shape=jax.ShapeDtypeStruct((1, VOCAB), jnp.float32),
        compiler_params=pltpu.CompilerParams(vmem_limit_bytes=100 * 1024 * 1024),
        scratch_shapes=[
            pltpu.VMEM((1, EMB), jnp.float32),
            pltpu.VMEM((1, BV), jnp.float32),
            pltpu.VMEM((1, BV), jnp.float32),
            pltpu.VMEM((NBLK, 1, BV), jnp.float32),
            pltpu.SMEM((2,), jnp.float32),
        ],
    )(c0, c1, tT, wT, b2)


def kernel(indices, emb_table, W, b):
    idx = indices.astype(jnp.int32)
    c0, c1 = _hist()(idx)
    return _tc_call(
        c0.reshape(1, VOCAB_PAD),
        c1.reshape(1, VOCAB_PAD),
        emb_table.T,
        W.T,
        b.reshape(1, VOCAB),
    )


# speedup vs baseline: 1.0116x; 1.0116x over previous
import functools

import jax
import jax.numpy as jnp
from jax import lax
from jax.experimental import pallas as pl
from jax.experimental.pallas import tpu as pltpu
from jax.experimental.pallas import tpu_sc as plsc

VOCAB = 100000
EMB = 64
N_CTX = 16384

NW = 32
PER_W = N_CTX // NW
CHUNK = 128
NCHUNK = PER_W // CHUNK

VOCAB_PAD = 100352
STRIPE = VOCAB_PAD // 16

BV = 25088
NBLK = VOCAB_PAD // BV


def _hist_body(idx_hbm, out0_hbm, out1_hbm, idx_v, zbuf, ones_v, shared, sem):
    core = lax.axis_index("c")
    sub = lax.axis_index("s")
    wid = sub * 2 + core

    idx_copies = [
        pltpu.async_copy(
            idx_hbm.at[pl.ds(wid * PER_W + k * CHUNK, CHUNK)], idx_v.at[k], sem
        )
        for k in range(NCHUNK)
    ]

    z16 = jnp.zeros((16,), jnp.float32)

    def zero_body(j, _):
        for u in range(8):
            zbuf[pl.ds(j * 128 + u * 16, 16)] = z16
        return 0

    lax.fori_loop(0, STRIPE // 128, zero_body, 0)
    for j in range(CHUNK // 16):
        ones_v[pl.ds(j * 16, 16)] = z16 + 1.0
    pltpu.sync_copy(zbuf, shared.at[pl.ds(sub * STRIPE, STRIPE)])
    for c in idx_copies:
        c.wait()
    plsc.subcore_barrier()
    for k in range(NCHUNK):
        pltpu.sync_copy(ones_v, shared.at[idx_v.at[k]], add=True)
    plsc.subcore_barrier()

    @pl.when(core == 0)
    def _out0():
        pltpu.sync_copy(
            shared.at[pl.ds(sub * STRIPE, STRIPE)],
            out0_hbm.at[pl.ds(sub * STRIPE, STRIPE)],
        )

    @pl.when(core == 1)
    def _out1():
        pltpu.sync_copy(
            shared.at[pl.ds(sub * STRIPE, STRIPE)],
            out1_hbm.at[pl.ds(sub * STRIPE, STRIPE)],
        )


@functools.cache
def _hist():
    return functools.partial(
        pl.kernel,
        out_type=[
            jax.ShapeDtypeStruct((VOCAB_PAD,), jnp.float32),
            jax.ShapeDtypeStruct((VOCAB_PAD,), jnp.float32),
        ],
        mesh=plsc.VectorSubcoreMesh(core_axis_name="c", subcore_axis_name="s"),
        scratch_types=[
            pltpu.VMEM((NCHUNK, CHUNK), jnp.int32),
            pltpu.VMEM((STRIPE,), jnp.float32),
            pltpu.VMEM((CHUNK,), jnp.float32),
            pltpu.VMEM_SHARED((VOCAB_PAD,), jnp.float32),
            pltpu.SemaphoreType.DMA,
        ],
    )(_hist_body)


def _tc_body(c0_ref, c1_ref, t_ref, w_ref, b_ref, out_ref, s_acc, mv, sv, logits_s, sm):
    p = pl.program_id(0)
    i = pl.program_id(1)
    lane = lax.broadcasted_iota(jnp.int32, (1, BV), 1)
    nvalid = VOCAB - i * BV

    @pl.when((p == 0) & (i == 0))
    def _init_s():
        s_acc[...] = jnp.zeros((1, EMB), jnp.float32)

    @pl.when((p == 0) & (i < NBLK - 1))
    def _phase0():
        c = c0_ref[...] + c1_ref[...]
        s_acc[...] += lax.dot_general(
            c, t_ref[...], (((1,), (1,)), ((), ())),
            preferred_element_type=jnp.float32,
        )

    @pl.when((p == 0) & (i == NBLK - 1))
    def _phase0_edge():
        c = c0_ref[...] + c1_ref[...]
        t = jnp.where(lane < nvalid, t_ref[...], 0.0)
        s_acc[...] += lax.dot_general(
            c, t, (((1,), (1,)), ((), ())),
            preferred_element_type=jnp.float32,
        )

    @pl.when((p == 1) & (i == 0))
    def _init_ms():
        mv[...] = jnp.full((1, BV), -1e30, jnp.float32)
        sv[...] = jnp.zeros((1, BV), jnp.float32)

    @pl.when(p == 1)
    def _phase1():
        s2 = s_acc[...] * (1.0 / N_CTX)
        l = lax.dot_general(
            s2, w_ref[...], (((1,), (0,)), ((), ())),
            preferred_element_type=jnp.float32,
        ) + b_ref[...]
        l = jnp.where(lane < nvalid, l, -1e30)
        logits_s[i] = l
        m_old = mv[...]
        m_new = jnp.maximum(m_old, l)
        sv[...] = sv[...] * jnp.exp(m_old - m_new) + jnp.exp(l - m_new)
        mv[...] = m_new

    @pl.when((p == 2) & (i == 0))
    def _logz():
        m = jnp.max(mv[...])
        sm[0] = m + jnp.log(jnp.sum(sv[...] * jnp.exp(mv[...] - m)))

    @pl.when(p == 2)
    def _phase2():
        out_ref[...] = logits_s[i] - sm[0]


def _tc_call(c0, c1, tT, wT, b2):
    last = NBLK - 1
    return pl.pallas_call(
        _tc_body,
        grid=(3, NBLK),
        in_specs=[
            pl.BlockSpec((1, BV), lambda p, i: (0, jnp.where(p == 0, i, last))),
            pl.BlockSpec((1, BV), lambda p, i: (0, jnp.where(p == 0, i, last))),
            pl.BlockSpec((EMB, BV), lambda p, i: (0, jnp.where(p == 0, i, last))),
            pl.BlockSpec((EMB, BV), lambda p, i: (0, jnp.where(p == 1, i, jnp.where(p == 0, 0, last)))),
            pl.BlockSpec((1, BV), lambda p, i: (0, jnp.where(p == 1, i, 0))),
        ],
        out_specs=pl.BlockSpec((1, BV), lambda p, i: (0, jnp.where(p == 2, i, 0))),
        out_shape=jax.ShapeDtypeStruct((1, VOCAB), jnp.float32),
        compiler_params=pltpu.CompilerParams(vmem_limit_bytes=100 * 1024 * 1024),
        scratch_shapes=[
            pltpu.VMEM((1, EMB), jnp.float32),
            pltpu.VMEM((1, BV), jnp.float32),
            pltpu.VMEM((1, BV), jnp.float32),
            pltpu.VMEM((NBLK, 1, BV), jnp.float32),
            pltpu.SMEM((2,), jnp.float32),
        ],
    )(c0, c1, tT, wT, b2)


def kernel(indices, emb_table, W, b):
    idx = indices.astype(jnp.int32)
    c0, c1 = _hist()(idx)
    return _tc_call(
        c0.reshape(1, VOCAB_PAD),
        c1.reshape(1, VOCAB_PAD),
        emb_table.T,
        W.T,
        b.reshape(1, VOCAB),
    )
